# baseline (speedup 1.0000x reference)
import functools
import jax
import jax.numpy as jnp
from jax import lax
from jax.experimental import pallas as pl
from jax.experimental.pallas import tpu as pltpu
from jax.experimental.pallas import tpu_sc as plsc

NODES = 40962
NIN = 163842
F = 256
NB = 16
NB7 = NB * 7
NWORKERS = 32
CHUNKS = NODES // NB
CPW = CHUNKS // NWORKERS
TAIL = NODES - CHUNKS * NB
TAIL7 = TAIL * 7
IDXPAD = NODES * 7 + NB7


def _pool_kernel(x_hbm, idx_hbm, out_hbm, idx0, idx1, g0, g1, out_v, sem0, sem1):
    wid = lax.axis_index("s") * 2 + lax.axis_index("c")
    lane = lax.broadcasted_iota(jnp.int32, (16,), 0)
    zero16 = jnp.zeros((16,), jnp.int32)
    lw = lax.shift_right_logical(lane * 7, 1)
    even_lane = lax.bitwise_and(lane, 1) == 0
    himask = jnp.full((16,), -65536, jnp.int32)
    base_ci = wid * CPW
    idxs = (idx0, idx1)
    gs = (g0, g1)
    sems = (sem0, sem1)

    def start_gather(ci, p):
        pltpu.sync_copy(idx_hbm.at[pl.ds(ci * NB7, NB7)], idxs[p])
        pltpu.async_copy(x_hbm.at[idxs[p]], gs[p], sems[p])

    def wait_gather(p):
        pltpu.make_async_copy(x_hbm.at[idxs[p]], gs[p], sems[p]).wait()

    def pooled_block(g_v, nodes, ci):
        @plsc.parallel_loop(0, nodes * 16, unroll=8)
        def fj_loop(i2):
            wb = lw + i2 * 56
            w0 = plsc.load_gather(g_v, [zero16, wb])
            w1 = plsc.load_gather(g_v, [zero16, wb + 1])
            w2 = plsc.load_gather(g_v, [zero16, wb + 2])
            w3 = plsc.load_gather(g_v, [zero16, wb + 3])
            lo0 = plsc.bitcast(lax.shift_left(w0, 16), jnp.float32)
            hi3 = plsc.bitcast(lax.bitwise_and(w3, himask), jnp.float32)
            s = (
                lo0
                + plsc.bitcast(lax.bitwise_and(w0, himask), jnp.float32)
                + plsc.bitcast(lax.shift_left(w1, 16), jnp.float32)
                + plsc.bitcast(lax.bitwise_and(w1, himask), jnp.float32)
                + plsc.bitcast(lax.shift_left(w2, 16), jnp.float32)
                + plsc.bitcast(lax.bitwise_and(w2, himask), jnp.float32)
                + plsc.bitcast(lax.shift_left(w3, 16), jnp.float32)
                + hi3
            )
            corr = jnp.where(even_lane, hi3, lo0)
            b = lax.shift_right_logical(i2, 4)
            j = lax.bitwise_and(i2, 15)
            out_v[b, pl.ds(j * 16, 16)] = (s - corr) * jnp.float32(1.0 / 7.0)

        pltpu.sync_copy(
            out_v.at[pl.ds(0, nodes)], out_hbm.at[pl.ds(ci * NB, nodes)]
        )

    start_gather(base_ci, 0)

    def pair_body(i, carry):
        ci = base_ci + 2 * i
        start_gather(ci + 1, 1)
        wait_gather(0)
        pooled_block(g0, NB, ci)
        start_gather(ci + 2, 0)
        wait_gather(1)
        pooled_block(g1, NB, ci + 1)
        return carry

    lax.fori_loop(0, CPW // 2, pair_body, 0)
    wait_gather(0)

    @pl.when(wid == NWORKERS - 1)
    def _():
        pltpu.sync_copy(
            idx_hbm.at[pl.ds(CHUNKS * NB7, TAIL7)], idx0.at[pl.ds(0, TAIL7)]
        )
        pltpu.async_copy(
            x_hbm.at[idx0.at[pl.ds(0, TAIL7)]], g0.at[pl.ds(0, TAIL7)], sem0
        ).wait()
        pooled_block(g0, TAIL, CHUNKS)


@jax.jit
def _pool(x, idx):
    mesh = plsc.VectorSubcoreMesh(core_axis_name="c", subcore_axis_name="s")
    kfn = functools.partial(
        pl.kernel,
        mesh=mesh,
        out_type=jax.ShapeDtypeStruct((NODES, F), jnp.float32),
        scratch_types=[
            pltpu.VMEM((NB7,), jnp.int32),
            pltpu.VMEM((NB7,), jnp.int32),
            pltpu.VMEM((NB7, F // 2), jnp.int32),
            pltpu.VMEM((NB7, F // 2), jnp.int32),
            pltpu.VMEM((NB, F), jnp.float32),
            pltpu.SemaphoreType.DMA,
            pltpu.SemaphoreType.DMA,
        ],
        compiler_params=pltpu.CompilerParams(
            use_tc_tiling_on_sc=False, needs_layout_passes=False
        ),
    )(_pool_kernel)
    return kfn(x, idx)


def kernel(x, neigh_orders):
    idx = neigh_orders.astype(jnp.int32)
    idx = jnp.pad(idx, (0, IDXPAD - idx.shape[0]))
    xr = x.reshape(NIN, F // 2, 2)
    lo = lax.bitcast_convert_type(
        xr[:, :, 0].astype(jnp.bfloat16), jnp.uint16
    ).astype(jnp.uint32)
    hi = lax.bitcast_convert_type(
        xr[:, :, 1].astype(jnp.bfloat16), jnp.uint16
    ).astype(jnp.uint32)
    xw = lax.bitcast_convert_type(lo | (hi << 16), jnp.int32)
    return _pool(xw, idx)


# speedup vs baseline: 2.7249x; 2.7249x over previous
import functools
import jax
import jax.numpy as jnp
from jax import lax
from jax.experimental import pallas as pl
from jax.experimental.pallas import tpu as pltpu
from jax.experimental.pallas import tpu_sc as plsc

NODES = 40962
NIN = 163842
F = 256
NB = 16
NB7 = NB * 7
NWORKERS = 32
CHUNKS = NODES // NB
CPW = CHUNKS // NWORKERS
TAIL = NODES - CHUNKS * NB
TAIL7 = TAIL * 7
IDXPAD = NODES * 7 + NB7


def _pool_kernel(x_hbm, idx_hbm, out_hbm, idx0, idx1, g0, g1, out_v, sem0, sem1):
    wid = lax.axis_index("s") * 2 + lax.axis_index("c")
    lane = lax.broadcasted_iota(jnp.int32, (16,), 0)
    zero16 = jnp.zeros((16,), jnp.int32)
    l7 = lane * 7
    base_ci = wid * CPW
    idxs = (idx0, idx1)
    gs = (g0, g1)
    sems = (sem0, sem1)

    def start_gather(ci, p):
        pltpu.sync_copy(idx_hbm.at[pl.ds(ci * NB7, NB7)], idxs[p])
        pltpu.async_copy(x_hbm.at[idxs[p]], gs[p], sems[p])

    def wait_gather(p):
        pltpu.make_async_copy(x_hbm.at[idxs[p]], gs[p], sems[p]).wait()

    def pooled_block(g_v, nodes, ci):
        @plsc.parallel_loop(0, nodes * 16, unroll=8)
        def fj_loop(i2):
            base = l7 + i2 * 112
            acc0 = plsc.load_gather(g_v, [zero16, base])
            acc1 = plsc.load_gather(g_v, [zero16, base + 1])
            acc2 = plsc.load_gather(g_v, [zero16, base + 2])
            acc0 = acc0 + plsc.load_gather(g_v, [zero16, base + 3])
            acc1 = acc1 + plsc.load_gather(g_v, [zero16, base + 4])
            acc2 = acc2 + plsc.load_gather(g_v, [zero16, base + 5])
            acc0 = acc0 + plsc.load_gather(g_v, [zero16, base + 6])
            b = lax.shift_right_logical(i2, 4)
            j = lax.bitwise_and(i2, 15)
            out_v[b, pl.ds(j * 16, 16)] = (acc0 + acc1 + acc2) * jnp.float32(
                1.0 / 7.0
            )

        pltpu.sync_copy(
            out_v.at[pl.ds(0, nodes)], out_hbm.at[pl.ds(ci * NB, nodes)]
        )

    start_gather(base_ci, 0)

    def pair_body(i, carry):
        ci = base_ci + 2 * i
        start_gather(ci + 1, 1)
        wait_gather(0)
        pooled_block(g0, NB, ci)
        start_gather(ci + 2, 0)
        wait_gather(1)
        pooled_block(g1, NB, ci + 1)
        return carry

    lax.fori_loop(0, CPW // 2, pair_body, 0)
    wait_gather(0)

    @pl.when(wid == NWORKERS - 1)
    def _():
        pltpu.sync_copy(
            idx_hbm.at[pl.ds(CHUNKS * NB7, TAIL7)], idx0.at[pl.ds(0, TAIL7)]
        )
        pltpu.async_copy(
            x_hbm.at[idx0.at[pl.ds(0, TAIL7)]], g0.at[pl.ds(0, TAIL7)], sem0
        ).wait()
        pooled_block(g0, TAIL, CHUNKS)


@jax.jit
def _pool(x, idx):
    mesh = plsc.VectorSubcoreMesh(core_axis_name="c", subcore_axis_name="s")
    kfn = functools.partial(
        pl.kernel,
        mesh=mesh,
        out_type=jax.ShapeDtypeStruct((NODES, F), jnp.float32),
        scratch_types=[
            pltpu.VMEM((NB7,), jnp.int32),
            pltpu.VMEM((NB7,), jnp.int32),
            pltpu.VMEM((NB7, F), jnp.float32),
            pltpu.VMEM((NB7, F), jnp.float32),
            pltpu.VMEM((NB, F), jnp.float32),
            pltpu.SemaphoreType.DMA,
            pltpu.SemaphoreType.DMA,
        ],
        compiler_params=pltpu.CompilerParams(
            use_tc_tiling_on_sc=False, needs_layout_passes=False
        ),
    )(_pool_kernel)
    return kfn(x, idx)


def kernel(x, neigh_orders):
    idx = neigh_orders.astype(jnp.int32)
    idx = jnp.pad(idx, (0, IDXPAD - idx.shape[0]))
    return _pool(x, idx)
